# baseline (speedup 1.0000x reference)
import functools
import math

import jax
import jax.numpy as jnp
from jax import lax
from jax.experimental import pallas as pl
from jax.experimental.pallas import tpu as pltpu
from jax.experimental.pallas import tpu_sc as plsc

D_MODEL = 768
SEQ = 4096
BATCH = 4
TOKENS = BATCH * SEQ
SCALE = math.sqrt(float(D_MODEL))

_INFO = plsc.get_sparse_core_info()
NUM_WORKERS = _INFO.num_cores * _INFO.num_subcores
PPW = SEQ // NUM_WORKERS
CB = 16
NCP = PPW // CB
NITER = NCP * BATCH
VPR = D_MODEL // 16
RD = 5
PD = 2


def _pe_table(seq_len, d_model):
    pos = jnp.arange(seq_len, dtype=jnp.float32)[:, None]
    div = jnp.exp(
        jnp.arange(0, d_model, 2, dtype=jnp.float32)
        * (-jnp.log(10000.0) / d_model)
    )
    pe = jnp.zeros((seq_len, d_model), dtype=jnp.float32)
    pe = pe.at[:, 0::2].set(jnp.sin(pos * div))
    pe = pe.at[:, 1::2].set(jnp.cos(pos * div))
    return pe


def _sc_body(idx_hbm, table_hbm, pe_hbm, out_hbm, idx_v, *scratch):
    rows = scratch[0:RD]
    pes = scratch[RD:RD + PD]
    gsem = scratch[RD + PD:2 * RD + PD]
    psem = scratch[2 * RD + PD:2 * RD + 2 * PD]
    ssem = scratch[2 * RD + 2 * PD:3 * RD + 2 * PD]

    wid = lax.axis_index("s") * _INFO.num_cores + lax.axis_index("c")
    pos0 = wid * PPW
    for b in range(BATCH):
        pltpu.sync_copy(
            idx_hbm.at[pl.ds(b * SEQ + pos0, PPW)],
            idx_v.at[pl.ds(b * PPW, PPW)])

    gather = {}
    peload = {}
    store = {}

    def issue_gather(n):
        if n < NITER:
            if n >= RD:
                store[n - RD].wait()
            c, b = divmod(n, BATCH)
            gather[n] = pltpu.async_copy(
                table_hbm.at[idx_v.at[pl.ds(b * PPW + c * CB, CB)]],
                rows[n % RD], gsem[n % RD])

    def issue_pe(c):
        if c < NCP:
            peload[c] = pltpu.async_copy(
                pe_hbm.at[pl.ds(pos0 + c * CB, CB)],
                pes[c % PD], psem[c % PD])

    issue_pe(0)
    issue_pe(1)
    for n in range(RD - 1):
        issue_gather(n)

    for n in range(NITER):
        c, b = divmod(n, BATCH)
        issue_gather(n + RD - 1)
        if b == 0:
            peload[c].wait()
        gather[n].wait()
        rbuf = rows[n % RD]
        pbuf = pes[c % PD]

        def fma_row(i, carry):
            for j in range(VPR):
                sl = pl.ds(j * 16, 16)
                rbuf[i, sl] = rbuf[i, sl] * SCALE + pbuf[i, sl]
            return carry

        lax.fori_loop(0, CB, fma_row, 0)
        store[n] = pltpu.async_copy(
            rbuf, out_hbm.at[pl.ds(b * SEQ + pos0 + c * CB, CB)],
            ssem[n % RD])
        if b == BATCH - 1:
            issue_pe(c + PD)

    for n in range(NITER - RD, NITER):
        store[n].wait()


def kernel(x, table):
    idx = x.reshape(-1).astype(jnp.int32)
    pe = _pe_table(SEQ, D_MODEL)
    mesh = plsc.VectorSubcoreMesh(core_axis_name="c", subcore_axis_name="s")
    scratch = (
        [pltpu.VMEM((BATCH * PPW,), jnp.int32)]
        + [pltpu.VMEM((CB, D_MODEL), jnp.float32) for _ in range(RD + PD)]
        + [pltpu.SemaphoreType.DMA for _ in range(2 * RD + PD)]
    )
    run = functools.partial(
        pl.kernel,
        out_type=jax.ShapeDtypeStruct((TOKENS, D_MODEL), jnp.float32),
        mesh=mesh,
        scratch_types=scratch,
    )(_sc_body)
    out = run(idx, table, pe)
    return out.reshape(BATCH, SEQ, D_MODEL)


# speedup vs baseline: 1.1378x; 1.1378x over previous
import functools
import math

import jax
import jax.numpy as jnp
from jax import lax
from jax.experimental import pallas as pl
from jax.experimental.pallas import tpu as pltpu
from jax.experimental.pallas import tpu_sc as plsc

D_MODEL = 768
SEQ = 4096
BATCH = 4
TOKENS = BATCH * SEQ
SCALE = math.sqrt(float(D_MODEL))

_INFO = plsc.get_sparse_core_info()
NUM_WORKERS = _INFO.num_cores * _INFO.num_subcores
PPW = SEQ // NUM_WORKERS
CB = 16
NCP = PPW // CB
NITER = NCP * BATCH
VPR = D_MODEL // 16
RD = 8
GD = 4
PD = 2


def _pe_table(seq_len, d_model):
    pos = jnp.arange(seq_len, dtype=jnp.float32)[:, None]
    div = jnp.exp(
        jnp.arange(0, d_model, 2, dtype=jnp.float32)
        * (-jnp.log(10000.0) / d_model)
    )
    pe = jnp.zeros((seq_len, d_model), dtype=jnp.float32)
    pe = pe.at[:, 0::2].set(jnp.sin(pos * div))
    pe = pe.at[:, 1::2].set(jnp.cos(pos * div))
    return pe


def _sc_body(idx_hbm, table_hbm, pe_hbm, out_hbm, idx_v, *scratch):
    rows = scratch[0:RD]
    pes = scratch[RD:RD + PD]
    gsem = scratch[RD + PD:2 * RD + PD]
    psem = scratch[2 * RD + PD:2 * RD + 2 * PD]
    ssem = scratch[2 * RD + 2 * PD:3 * RD + 2 * PD]

    wid = lax.axis_index("s") * _INFO.num_cores + lax.axis_index("c")
    pos0 = wid * PPW
    for b in range(BATCH):
        pltpu.sync_copy(
            idx_hbm.at[pl.ds(b * SEQ + pos0, PPW)],
            idx_v.at[pl.ds(b * PPW, PPW)])

    gather = {}
    peload = {}
    store = {}

    def issue_gather(n):
        if n < NITER:
            if n >= RD:
                store[n - RD].wait()
            c, b = divmod(n, BATCH)
            gather[n] = pltpu.async_copy(
                table_hbm.at[idx_v.at[pl.ds(b * PPW + c * CB, CB)]],
                rows[n % RD], gsem[n % RD])

    def issue_pe(c):
        if c < NCP:
            peload[c] = pltpu.async_copy(
                pe_hbm.at[pl.ds(pos0 + c * CB, CB)],
                pes[c % PD], psem[c % PD])

    issue_pe(0)
    issue_pe(1)
    for n in range(GD):
        issue_gather(n)

    for n in range(NITER):
        c, b = divmod(n, BATCH)
        issue_gather(n + GD)
        if b == 0:
            peload[c].wait()
        gather[n].wait()
        rbuf = rows[n % RD]
        pbuf = pes[c % PD]

        def fma_row(i, carry):
            for j in range(VPR):
                sl = pl.ds(j * 16, 16)
                rbuf[i, sl] = rbuf[i, sl] * SCALE + pbuf[i, sl]
            return carry

        lax.fori_loop(0, CB, fma_row, 0)
        store[n] = pltpu.async_copy(
            rbuf, out_hbm.at[pl.ds(b * SEQ + pos0 + c * CB, CB)],
            ssem[n % RD])
        if b == BATCH - 1:
            issue_pe(c + PD)

    for n in range(NITER - RD, NITER):
        store[n].wait()


def kernel(x, table):
    idx = x.reshape(-1).astype(jnp.int32)
    pe = _pe_table(SEQ, D_MODEL)
    mesh = plsc.VectorSubcoreMesh(core_axis_name="c", subcore_axis_name="s")
    scratch = (
        [pltpu.VMEM((BATCH * PPW,), jnp.int32)]
        + [pltpu.VMEM((CB, D_MODEL), jnp.float32) for _ in range(RD + PD)]
        + [pltpu.SemaphoreType.DMA for _ in range(2 * RD + PD)]
    )
    run = functools.partial(
        pl.kernel,
        out_type=jax.ShapeDtypeStruct((TOKENS, D_MODEL), jnp.float32),
        mesh=mesh,
        scratch_types=scratch,
    )(_sc_body)
    out = run(idx, table, pe)
    return out.reshape(BATCH, SEQ, D_MODEL)
